# Initial kernel scaffold; baseline (speedup 1.0000x reference)
import functools

import jax
import jax.numpy as jnp
from jax import lax
from jax.experimental import pallas as pl
from jax.experimental.pallas import tpu as pltpu
from jax.experimental.pallas import tpu_sc as plsc

N = 320000
D = 128
NSYS = 1024

NUM_CORES = 2
NUM_SUBCORES = 16
NW = NUM_CORES * NUM_SUBCORES
RPW = N // NW
CHUNK = 80
NCHUNK = RPW // CHUNK
ROWS_PER_TILE_OUT = NSYS // NUM_SUBCORES

_zero16 = jnp.zeros((16,), jnp.float32)


def _sc_partial_sums(x_r, bi_r):
    mesh = plsc.VectorSubcoreMesh(core_axis_name="c", subcore_axis_name="s")

    @functools.partial(
        pl.kernel,
        out_type=jax.ShapeDtypeStruct((NUM_CORES, NSYS, D), jnp.float32),
        mesh=mesh,
        scratch_types=[
            pltpu.VMEM_SHARED((NSYS, D), jnp.float32),
            pltpu.VMEM((NCHUNK, CHUNK), jnp.int32),
            pltpu.VMEM((CHUNK, D), jnp.float32),
            pltpu.VMEM((ROWS_PER_TILE_OUT, D), jnp.float32),
        ],
    )
    def body(x_hbm, bi_hbm, part_hbm, acc, idx_v, rows_v, zb):
        c = lax.axis_index("c")
        s = lax.axis_index("s")
        w = c * NUM_SUBCORES + s

        def zrow(i, carry):
            for j in range(D // 16):
                zb[i, pl.ds(j * 16, 16)] = _zero16
            return carry

        lax.fori_loop(0, ROWS_PER_TILE_OUT, zrow, 0)
        pltpu.sync_copy(zb, acc.at[pl.ds(s * ROWS_PER_TILE_OUT, ROWS_PER_TILE_OUT)])
        plsc.subcore_barrier()

        pltpu.sync_copy(bi_hbm.at[w], idx_v)

        def chunk_body(j, carry):
            pltpu.sync_copy(x_hbm.at[w, j], rows_v)
            pltpu.sync_copy(rows_v, acc.at[idx_v.at[j]], add=True)
            return carry

        lax.fori_loop(0, NCHUNK, chunk_body, 0)
        plsc.subcore_barrier()

        pltpu.sync_copy(acc.at[pl.ds(s * ROWS_PER_TILE_OUT, ROWS_PER_TILE_OUT)], zb)
        pltpu.sync_copy(zb, part_hbm.at[c, pl.ds(s * ROWS_PER_TILE_OUT, ROWS_PER_TILE_OUT)])

    return body(x_r, bi_r)


def _combine_body(p_ref, o_ref):
    o_ref[...] = p_ref[0] + p_ref[1]


def kernel(x, batch_index, natoms):
    del natoms
    x_r = x.reshape(NW, NCHUNK, CHUNK, D)
    bi_r = batch_index.reshape(NW, NCHUNK, CHUNK)
    part = _sc_partial_sums(x_r, bi_r)
    out = pl.pallas_call(
        _combine_body,
        out_shape=jax.ShapeDtypeStruct((NSYS, D), jnp.float32),
    )(part)
    return out


# speedup vs baseline: 4.7718x; 4.7718x over previous
import functools

import jax
import jax.numpy as jnp
from jax import lax
from jax.experimental import pallas as pl
from jax.experimental.pallas import tpu as pltpu
from jax.experimental.pallas import tpu_sc as plsc

N = 320000
D = 128
NSYS = 1024

NUM_CORES = 2
NUM_SUBCORES = 16
NW = NUM_CORES * NUM_SUBCORES
RPW = N // NW
CHUNK = 80
NCHUNK = RPW // CHUNK
ROWS_PER_TILE_OUT = NSYS // NUM_SUBCORES

def _sc_partial_sums(x_r, bi_r):
    mesh = plsc.VectorSubcoreMesh(core_axis_name="c", subcore_axis_name="s")

    @functools.partial(
        pl.kernel,
        out_type=jax.ShapeDtypeStruct((NUM_CORES, NSYS, D), jnp.float32),
        mesh=mesh,
        scratch_types=[
            pltpu.VMEM_SHARED((NSYS, D), jnp.float32),
            pltpu.VMEM((NCHUNK, CHUNK), jnp.int32),
            pltpu.VMEM((CHUNK, D), jnp.float32),
            pltpu.VMEM((ROWS_PER_TILE_OUT, D), jnp.float32),
        ],
    )
    def body(x_hbm, bi_hbm, part_hbm, acc, idx_v, rows_v, zb):
        c = lax.axis_index("c")
        s = lax.axis_index("s")
        w = c * NUM_SUBCORES + s

        zero16 = jnp.zeros((16,), jnp.float32)

        def zrow(i, carry):
            for j in range(D // 16):
                zb[i, pl.ds(j * 16, 16)] = zero16
            return carry

        lax.fori_loop(0, ROWS_PER_TILE_OUT, zrow, 0)
        pltpu.sync_copy(zb, acc.at[pl.ds(s * ROWS_PER_TILE_OUT, ROWS_PER_TILE_OUT)])
        plsc.subcore_barrier()

        pltpu.sync_copy(bi_hbm.at[w], idx_v)

        def chunk_body(j, carry):
            pltpu.sync_copy(x_hbm.at[w, j], rows_v)
            pltpu.sync_copy(rows_v, acc.at[idx_v.at[j]], add=True)
            return carry

        lax.fori_loop(0, NCHUNK, chunk_body, 0)
        plsc.subcore_barrier()

        pltpu.sync_copy(acc.at[pl.ds(s * ROWS_PER_TILE_OUT, ROWS_PER_TILE_OUT)], zb)
        pltpu.sync_copy(zb, part_hbm.at[c, pl.ds(s * ROWS_PER_TILE_OUT, ROWS_PER_TILE_OUT)])

    return body(x_r, bi_r)


def _combine_body(p_ref, o_ref):
    o_ref[...] = p_ref[0] + p_ref[1]


def kernel(x, batch_index, natoms):
    del natoms
    x_r = x.reshape(NW, NCHUNK, CHUNK, D)
    bi_r = batch_index.reshape(NW, NCHUNK, CHUNK)
    part = _sc_partial_sums(x_r, bi_r)
    out = pl.pallas_call(
        _combine_body,
        out_shape=jax.ShapeDtypeStruct((NSYS, D), jnp.float32),
    )(part)
    return out


# speedup vs baseline: 7.3032x; 1.5305x over previous
import functools

import jax
import jax.numpy as jnp
from jax import lax
from jax.experimental import pallas as pl
from jax.experimental.pallas import tpu as pltpu
from jax.experimental.pallas import tpu_sc as plsc

N = 320000
D = 128
NSYS = 1024

NUM_CORES = 2
NUM_SUBCORES = 16
NW = NUM_CORES * NUM_SUBCORES
RPW = N // NW
CHUNK = 80
NCHUNK = RPW // CHUNK
NBUF = 5
ROWS_PER_TILE_OUT = NSYS // NUM_SUBCORES

def _sc_partial_sums(x_r, bi_r):
    mesh = plsc.VectorSubcoreMesh(core_axis_name="c", subcore_axis_name="s")

    @functools.partial(
        pl.kernel,
        out_type=jax.ShapeDtypeStruct((NUM_CORES, NSYS, D), jnp.float32),
        mesh=mesh,
        scratch_types=[
            pltpu.VMEM_SHARED((NSYS, D), jnp.float32),
            pltpu.VMEM((NCHUNK, CHUNK), jnp.int32),
            pltpu.VMEM((NBUF, CHUNK, D), jnp.float32),
            pltpu.VMEM((ROWS_PER_TILE_OUT, D), jnp.float32),
            pltpu.SemaphoreType.DMA((NBUF,)),
        ],
    )
    def body(x_hbm, bi_hbm, part_hbm, acc, idx_v, rows_v, zb, gsem):
        c = lax.axis_index("c")
        s = lax.axis_index("s")
        w = c * NUM_SUBCORES + s

        zero16 = jnp.zeros((16,), jnp.float32)

        def zrow(i, carry):
            for j in range(D // 16):
                zb[i, pl.ds(j * 16, 16)] = zero16
            return carry

        lax.fori_loop(0, ROWS_PER_TILE_OUT, zrow, 0)
        pltpu.sync_copy(zb, acc.at[pl.ds(s * ROWS_PER_TILE_OUT, ROWS_PER_TILE_OUT)])
        plsc.subcore_barrier()

        pltpu.sync_copy(bi_hbm.at[w], idx_v)

        for b in range(NBUF):
            pltpu.async_copy(x_hbm.at[w, b], rows_v.at[b], gsem.at[b])

        def group_body(g, carry):
            for b in range(NBUF):
                j = g * NBUF + b
                pltpu.make_async_copy(x_hbm.at[w, j], rows_v.at[b], gsem.at[b]).wait()
                pltpu.sync_copy(rows_v.at[b], acc.at[idx_v.at[j]], add=True)

                @pl.when(j + NBUF < NCHUNK)
                def _refill():
                    pltpu.async_copy(x_hbm.at[w, j + NBUF], rows_v.at[b], gsem.at[b])

            return carry

        lax.fori_loop(0, NCHUNK // NBUF, group_body, 0)
        plsc.subcore_barrier()

        pltpu.sync_copy(acc.at[pl.ds(s * ROWS_PER_TILE_OUT, ROWS_PER_TILE_OUT)], zb)
        pltpu.sync_copy(zb, part_hbm.at[c, pl.ds(s * ROWS_PER_TILE_OUT, ROWS_PER_TILE_OUT)])

    return body(x_r, bi_r)


def _combine_body(p_ref, o_ref):
    o_ref[...] = p_ref[0] + p_ref[1]


def kernel(x, batch_index, natoms):
    del natoms
    x_r = x.reshape(NW, NCHUNK, CHUNK, D)
    bi_r = batch_index.reshape(NW, NCHUNK, CHUNK)
    part = _sc_partial_sums(x_r, bi_r)
    out = pl.pallas_call(
        _combine_body,
        out_shape=jax.ShapeDtypeStruct((NSYS, D), jnp.float32),
    )(part)
    return out


# speedup vs baseline: 7.6275x; 1.0444x over previous
import functools

import jax
import jax.numpy as jnp
from jax import lax
from jax.experimental import pallas as pl
from jax.experimental.pallas import tpu as pltpu
from jax.experimental.pallas import tpu_sc as plsc

N = 320000
D = 128
NSYS = 1024

NUM_CORES = 2
NUM_SUBCORES = 16
NW = NUM_CORES * NUM_SUBCORES
RPW = N // NW
CHUNK = 80
NCHUNK = RPW // CHUNK
NBUF = 5
ROWS_PER_TILE_OUT = NSYS // NUM_SUBCORES

def _sc_partial_sums(x_r, bi_r):
    mesh = plsc.VectorSubcoreMesh(core_axis_name="c", subcore_axis_name="s")

    @functools.partial(
        pl.kernel,
        out_type=jax.ShapeDtypeStruct((NUM_CORES, NSYS, D), jnp.float32),
        mesh=mesh,
        scratch_types=[
            pltpu.VMEM_SHARED((NSYS, D), jnp.float32),
            pltpu.VMEM((NCHUNK, CHUNK), jnp.int32),
            pltpu.VMEM((NBUF, CHUNK, D), jnp.float32),
            pltpu.VMEM((ROWS_PER_TILE_OUT, D), jnp.float32),
            pltpu.SemaphoreType.DMA((NBUF,)),
            pltpu.SemaphoreType.DMA((NBUF,)),
        ],
    )
    def body(x_hbm, bi_hbm, part_hbm, acc, idx_v, rows_v, zb, gsem, ssem):
        c = lax.axis_index("c")
        s = lax.axis_index("s")
        w = c * NUM_SUBCORES + s

        zero16 = jnp.zeros((16,), jnp.float32)

        def zrow(i, carry):
            for j in range(D // 16):
                zb[i, pl.ds(j * 16, 16)] = zero16
            return carry

        lax.fori_loop(0, ROWS_PER_TILE_OUT, zrow, 0)
        pltpu.sync_copy(zb, acc.at[pl.ds(s * ROWS_PER_TILE_OUT, ROWS_PER_TILE_OUT)])
        plsc.subcore_barrier()

        pltpu.sync_copy(bi_hbm.at[w], idx_v)

        LEAD = NBUF - 2

        def wait_gather(j, b):
            pltpu.make_async_copy(x_hbm.at[w, j], rows_v.at[b], gsem.at[b]).wait()

        def wait_scatter(j, b):
            pltpu.make_async_copy(rows_v.at[b], acc.at[idx_v.at[j]], ssem.at[b]).wait()

        for b in range(LEAD):
            pltpu.async_copy(x_hbm.at[w, b], rows_v.at[b], gsem.at[b])

        def iter_body(j, carry):
            b = lax.rem(j, NBUF)
            wait_gather(j, b)
            pltpu.async_copy(rows_v.at[b], acc.at[idx_v.at[j]], ssem.at[b], add=True)
            jn = j + LEAD
            bn = lax.rem(jn, NBUF)

            @pl.when(jn < NCHUNK)
            def _refill():
                @pl.when(jn >= NBUF)
                def _drain():
                    wait_scatter(jn - NBUF, bn)

                pltpu.async_copy(x_hbm.at[w, jn], rows_v.at[bn], gsem.at[bn])

            return carry

        lax.fori_loop(0, NCHUNK, iter_body, 0)

        for k in range(NBUF):
            j_last = NCHUNK - NBUF + k
            wait_scatter(j_last, j_last % NBUF)
        plsc.subcore_barrier()

        pltpu.sync_copy(acc.at[pl.ds(s * ROWS_PER_TILE_OUT, ROWS_PER_TILE_OUT)], zb)
        pltpu.sync_copy(zb, part_hbm.at[c, pl.ds(s * ROWS_PER_TILE_OUT, ROWS_PER_TILE_OUT)])

    return body(x_r, bi_r)


def _combine_body(p_ref, o_ref):
    o_ref[...] = p_ref[0] + p_ref[1]


def kernel(x, batch_index, natoms):
    del natoms
    x_r = x.reshape(NW, NCHUNK, CHUNK, D)
    bi_r = batch_index.reshape(NW, NCHUNK, CHUNK)
    part = _sc_partial_sums(x_r, bi_r)
    out = pl.pallas_call(
        _combine_body,
        out_shape=jax.ShapeDtypeStruct((NSYS, D), jnp.float32),
    )(part)
    return out
